# baseline (speedup 1.0000x reference)
import functools

import jax
import jax.numpy as jnp
from jax import lax
from jax.experimental import pallas as pl
from jax.experimental.pallas import tpu as pltpu
from jax.experimental.pallas import tpu_sc as plsc

N_NODES = 10000
NPAD = 10240
N_EDGES = 320000
NC = 2
NS = 16
NR = 5120
ACC_ROWS = 5376
ART = ACC_ROWS // NS
CHUNK = 128
NCH = 157
EPT = NCH * CHUNK
EPAD = NS * EPT
RPT = NPAD // NS
BN = 256
NB = NPAD // BN
NBR = NR // BN


@functools.cache
def _mesh():
    return plsc.VectorSubcoreMesh(core_axis_name="c", subcore_axis_name="s",
                                  num_cores=NC, num_subcores=NS)


def _agg_pass(hs_hbm, zero_hbm, out_slc, p, src_slc, dst_slc, cnt_hbm,
              src_v, dst_v, rows0_v, rows1_v, cnt_v, acc_sh, sem0, sem1,
              barrier_before, trip_split=None):
    s = lax.axis_index("s")
    if barrier_before:
        plsc.subcore_barrier()
    pltpu.sync_copy(zero_hbm, acc_sh.at[pl.ds(s * ART, ART)])
    pltpu.sync_copy(src_slc, src_v)
    pltpu.sync_copy(dst_slc, dst_v)
    pltpu.sync_copy(cnt_hbm.at[s], cnt_v)
    cnt = jnp.max(cnt_v[pl.ds(p * 16, 16)])
    trips = (cnt + CHUNK - 1) // CHUNK
    if trip_split is None:
        lo, hi = jnp.int32(0), trips
    else:
        half = trips // 2
        lo = jnp.where(trip_split == 0, 0, half)
        hi = jnp.where(trip_split == 0, half, trips)
    plsc.subcore_barrier()

    @pl.when(hi > lo)
    def _():
        pltpu.async_copy(hs_hbm.at[src_v.at[lo]], rows0_v, sem0)

    def body(j, carry):
        @pl.when((j - lo) % 2 == 0)
        def _():
            @pl.when(j + 1 < hi)
            def _():
                pltpu.async_copy(hs_hbm.at[src_v.at[j + 1]], rows1_v, sem1)
            pltpu.make_async_copy(hs_hbm.at[src_v.at[j]],
                                  rows0_v, sem0).wait()
            pltpu.sync_copy(rows0_v, acc_sh.at[dst_v.at[j]], add=True)

        @pl.when((j - lo) % 2 == 1)
        def _():
            @pl.when(j + 1 < hi)
            def _():
                pltpu.async_copy(hs_hbm.at[src_v.at[j + 1]], rows0_v, sem0)
            pltpu.make_async_copy(hs_hbm.at[src_v.at[j]],
                                  rows1_v, sem1).wait()
            pltpu.sync_copy(rows1_v, acc_sh.at[dst_v.at[j]], add=True)

        return carry

    lax.fori_loop(lo, hi, body, 0)
    plsc.subcore_barrier()
    pltpu.sync_copy(acc_sh.at[pl.ds(s * ART, ART)], out_slc)


def _agg_scratch():
    return [
        pltpu.VMEM((NCH, CHUNK), jnp.int32),
        pltpu.VMEM((NCH, CHUNK), jnp.int32),
        pltpu.VMEM((CHUNK, 128), jnp.float32),
        pltpu.VMEM((CHUNK, 128), jnp.float32),
        pltpu.VMEM((32,), jnp.int32),
        pltpu.VMEM_SHARED((ACC_ROWS, 128), jnp.float32),
        pltpu.SemaphoreType.DMA,
        pltpu.SemaphoreType.DMA,
    ]


@functools.cache
def _make_agg1():
    @functools.partial(
        pl.kernel,
        out_type=jax.ShapeDtypeStruct((NC, 2, ACC_ROWS, 128), jnp.float32),
        mesh=_mesh(),
        compiler_params=pltpu.CompilerParams(needs_layout_passes=False),
        scratch_types=_agg_scratch(),
    )
    def agg1(hs_hbm, src_hbm, dst_hbm, cnt_hbm, zero_hbm, out_hbm,
             src_v, dst_v, rows0_v, rows1_v, cnt_v, acc_sh, sem0, sem1):
        c = lax.axis_index("c")
        s = lax.axis_index("s")
        for p in range(2):
            _agg_pass(hs_hbm, zero_hbm,
                      out_hbm.at[c, p, pl.ds(s * ART, ART)],
                      p, src_hbm.at[c, p, s], dst_hbm.at[p, s], cnt_hbm,
                      src_v, dst_v, rows0_v, rows1_v, cnt_v, acc_sh,
                      sem0, sem1, barrier_before=(p > 0))

    return agg1


@functools.cache
def _make_agg2():
    @functools.partial(
        pl.kernel,
        out_type=jax.ShapeDtypeStruct((NC, 2, ACC_ROWS, 128), jnp.float32),
        mesh=_mesh(),
        compiler_params=pltpu.CompilerParams(needs_layout_passes=False),
        scratch_types=_agg_scratch(),
    )
    def agg2(hs_hbm, src_hbm, dst_hbm, cnt_hbm, zero_hbm, out_hbm,
             src_v, dst_v, rows0_v, rows1_v, cnt_v, acc_sh, sem0, sem1):
        c = lax.axis_index("c")
        s = lax.axis_index("s")
        for p in range(2):
            _agg_pass(hs_hbm, zero_hbm,
                      out_hbm.at[c, p, pl.ds(s * ART, ART)],
                      p, src_hbm.at[p, s], dst_hbm.at[p, s], cnt_hbm,
                      src_v, dst_v, rows0_v, rows1_v, cnt_v, acc_sh,
                      sem0, sem1, barrier_before=(p > 0), trip_split=c)

    return agg2


@functools.cache
def _make_part():
    @functools.partial(
        pl.kernel,
        out_type=(jax.ShapeDtypeStruct((2, NS, EPT), jnp.int32),
                  jax.ShapeDtypeStruct((2, NS, EPT), jnp.int32),
                  jax.ShapeDtypeStruct((NS, 32), jnp.int32)),
        mesh=_mesh(),
        compiler_params=pltpu.CompilerParams(needs_layout_passes=False),
        scratch_types=[
            pltpu.VMEM((EPT,), jnp.int32),
            pltpu.VMEM((EPT,), jnp.int32),
            pltpu.VMEM((EPT,), jnp.int32),
            pltpu.VMEM((EPT,), jnp.int32),
            pltpu.VMEM((EPT,), jnp.int32),
            pltpu.VMEM((EPT,), jnp.int32),
            pltpu.VMEM((32,), jnp.int32),
        ],
    )
    def part_kernel(src_hbm, dst_hbm, zfill_hbm, dfill_hbm,
                    osrc_hbm, odst_hbm, ocnt_hbm,
                    src_v, dst_v, bs0, bs1, bd0, bd1, cnt_v):
        c = lax.axis_index("c")
        s = lax.axis_index("s")

        @pl.when(c == 0)
        def _():
            pltpu.sync_copy(src_hbm.at[s], src_v)
            pltpu.sync_copy(dst_hbm.at[s], dst_v)
            pltpu.sync_copy(zfill_hbm, bs0)
            pltpu.sync_copy(zfill_hbm, bs1)
            pltpu.sync_copy(dfill_hbm, bd0)
            pltpu.sync_copy(dfill_hbm, bd1)

            def body(i, cnts):
                c0, c1 = cnts
                sv = src_v[pl.ds(i * 16, 16)]
                dv = dst_v[pl.ds(i * 16, 16)]
                m0 = dv < NR
                m1 = jnp.logical_not(m0)
                plsc.store_compressed(bs0.at[pl.ds(c0, 16)], sv, mask=m0)
                plsc.store_compressed(bd0.at[pl.ds(c0, 16)], dv, mask=m0)
                plsc.store_compressed(bs1.at[pl.ds(c1, 16)], sv, mask=m1)
                plsc.store_compressed(bd1.at[pl.ds(c1, 16)], dv - NR,
                                      mask=m1)
                n0 = jnp.sum(m0.astype(jnp.int32))
                return (c0 + n0, c1 + (16 - n0))

            c0, c1 = lax.fori_loop(0, EPT // 16, body,
                                   (jnp.int32(0), jnp.int32(0)))
            cnt_v[pl.ds(0, 16)] = jnp.full((16,), 1, jnp.int32) * c0
            cnt_v[pl.ds(16, 16)] = jnp.full((16,), 1, jnp.int32) * c1
            pltpu.sync_copy(bs0, osrc_hbm.at[0, s])
            pltpu.sync_copy(bs1, osrc_hbm.at[1, s])
            pltpu.sync_copy(bd0, odst_hbm.at[0, s])
            pltpu.sync_copy(bd1, odst_hbm.at[1, s])
            pltpu.sync_copy(cnt_v, ocnt_hbm.at[s])

    return part_kernel


@functools.cache
def _make_deg():
    @functools.partial(
        pl.kernel,
        out_type=jax.ShapeDtypeStruct((NPAD,), jnp.float32),
        mesh=_mesh(),
        compiler_params=pltpu.CompilerParams(needs_layout_passes=False),
        scratch_types=[
            pltpu.VMEM((EPT,), jnp.int32),
            pltpu.VMEM((NPAD,), jnp.float32),
            pltpu.VMEM((NS, RPT), jnp.float32),
            pltpu.VMEM((RPT,), jnp.float32),
            pltpu.VMEM_SHARED((NS, NPAD), jnp.float32),
        ],
    )
    def deg_kernel(dst_hbm, zero_hbm, out_hbm, dst_v, ldeg_v, mrg_v, mout_v,
                   sh):
        c = lax.axis_index("c")
        s = lax.axis_index("s")

        @pl.when(c == 0)
        def _():
            pltpu.sync_copy(dst_hbm.at[s], dst_v)
            pltpu.sync_copy(zero_hbm, ldeg_v)
            ones16 = jnp.full((16,), 1.0, jnp.float32)

            def body(i, carry):
                idx = dst_v[pl.ds(i * 16, 16)]
                plsc.addupdate_scatter(ldeg_v, [idx], ones16)
                return carry

            lax.fori_loop(0, EPT // 16, body, 0)
            pltpu.sync_copy(ldeg_v, sh.at[s])
            plsc.subcore_barrier()
            pltpu.sync_copy(sh.at[:, pl.ds(s * RPT, RPT)], mrg_v)
            for k in range(RPT // 16):
                acc = None
                for t in range(NS):
                    row = mrg_v[t, pl.ds(k * 16, 16)]
                    acc = row if acc is None else acc + row
                mout_v[pl.ds(k * 16, 16)] = acc
            pltpu.sync_copy(mout_v, out_hbm.at[pl.ds(s * RPT, RPT)])

    return deg_kernel


def _tc1(x_p, w1h, deg):
    def body(x_ref, w_ref, deg_ref, o_ref):
        dis = lax.rsqrt(deg_ref[...] + 1.0)
        h = jnp.dot(x_ref[...], w_ref[0], preferred_element_type=jnp.float32)
        o_ref[...] = h * dis

    return pl.pallas_call(
        body,
        grid=(NB, 2),
        in_specs=[
            pl.BlockSpec((BN, 128), lambda i, j: (i, 0)),
            pl.BlockSpec((1, 128, 128), lambda i, j: (j, 0, 0)),
            pl.BlockSpec((BN, 1), lambda i, j: (i, 0)),
        ],
        out_specs=pl.BlockSpec((BN, 128), lambda i, j: (j * NB + i, 0)),
        out_shape=jax.ShapeDtypeStruct((2 * NPAD, 128), jnp.float32),
    )(x_p, w1h, deg)


def _tc2(agg1, hs1, deg, b1, w2):
    def body(a_lo, a_hi, h_lo, h_hi, deg_ref, b_ref, w_ref, o_ref):
        dis = lax.rsqrt(deg_ref[...] + 1.0)
        b = b_ref[...]
        w = w_ref[...]
        r_lo = jnp.maximum(dis * (a_lo[0, 0] + h_lo[...]) + b[:, :128], 0.0)
        r_hi = jnp.maximum(dis * (a_hi[0, 0] + h_hi[...]) + b[:, 128:], 0.0)
        h2 = (jnp.dot(r_lo, w[:128], preferred_element_type=jnp.float32)
              + jnp.dot(r_hi, w[128:], preferred_element_type=jnp.float32))
        o_ref[...] = h2 * dis

    return pl.pallas_call(
        body,
        grid=(NB,),
        in_specs=[
            pl.BlockSpec((1, 1, BN, 128), lambda i: (0, i // NBR, i % NBR, 0)),
            pl.BlockSpec((1, 1, BN, 128), lambda i: (1, i // NBR, i % NBR, 0)),
            pl.BlockSpec((BN, 128), lambda i: (i, 0)),
            pl.BlockSpec((BN, 128), lambda i: (NB + i, 0)),
            pl.BlockSpec((BN, 1), lambda i: (i, 0)),
            pl.BlockSpec((1, 256), lambda i: (0, 0)),
            pl.BlockSpec((256, 128), lambda i: (0, 0)),
        ],
        out_specs=pl.BlockSpec((BN, 128), lambda i: (i, 0)),
        out_shape=jax.ShapeDtypeStruct((NPAD, 128), jnp.float32),
    )(agg1, agg1, hs1, hs1, deg, b1, w2)


def _tc3(agg2, hs2, deg, b2):
    def body(a0_ref, a1_ref, h_ref, deg_ref, b_ref, o_ref):
        dis = lax.rsqrt(deg_ref[...] + 1.0)
        o_ref[...] = (dis * (a0_ref[0, 0] + a1_ref[0, 0] + h_ref[...])
                      + b_ref[...])

    return pl.pallas_call(
        body,
        grid=(NB,),
        in_specs=[
            pl.BlockSpec((1, 1, BN, 128), lambda i: (0, i // NBR, i % NBR, 0)),
            pl.BlockSpec((1, 1, BN, 128), lambda i: (1, i // NBR, i % NBR, 0)),
            pl.BlockSpec((BN, 128), lambda i: (i, 0)),
            pl.BlockSpec((BN, 1), lambda i: (i, 0)),
            pl.BlockSpec((1, 128), lambda i: (0, 0)),
        ],
        out_specs=pl.BlockSpec((BN, 128), lambda i: (i, 0)),
        out_shape=jax.ShapeDtypeStruct((NPAD, 128), jnp.float32),
    )(agg2, agg2, hs2, deg, b2)


def kernel(x, edge_index, W1, b1, W2, b2):
    x = x.astype(jnp.float32)
    src = edge_index[0].astype(jnp.int32)
    dst = edge_index[1].astype(jnp.int32)
    pad_e = EPAD - N_EDGES
    srcp = jnp.concatenate([src, jnp.zeros((pad_e,), jnp.int32)])
    dstp = jnp.concatenate([dst, jnp.full((pad_e,), N_NODES, jnp.int32)])
    x_p = jnp.pad(x, ((0, NPAD - N_NODES), (0, 0)))
    zero_acc = jnp.zeros((ART, 128), jnp.float32)

    src_l, dst_l, counts = _make_part()(
        srcp.reshape(NS, EPT), dstp.reshape(NS, EPT),
        jnp.zeros((EPT,), jnp.int32), jnp.full((EPT,), NR, jnp.int32))
    src1 = (src_l[None] + jnp.array([0, NPAD], jnp.int32)[:, None, None, None]
            ).reshape(NC, 2, NS, NCH, CHUNK)
    src2 = src_l.reshape(2, NS, NCH, CHUNK)
    dst4 = dst_l.reshape(2, NS, NCH, CHUNK)

    deg = _make_deg()(dstp.reshape(NS, EPT),
                      jnp.zeros((NPAD,), jnp.float32))[:, None]
    hs1 = _tc1(x_p, jnp.stack([W1[:, :128], W1[:, 128:]]), deg)
    agg1 = _make_agg1()(hs1, src1, dst4, counts, zero_acc)
    hs2 = _tc2(agg1, hs1, deg, b1.reshape(1, 256), W2)
    agg2 = _make_agg2()(hs2, src2, dst4, counts, zero_acc)
    out = _tc3(agg2, hs2, deg, b2.reshape(1, 128))
    return out[:N_NODES]


# speedup vs baseline: 1.0188x; 1.0188x over previous
import functools

import jax
import jax.numpy as jnp
from jax import lax
from jax.experimental import pallas as pl
from jax.experimental.pallas import tpu as pltpu
from jax.experimental.pallas import tpu_sc as plsc

N_NODES = 10000
NPAD = 10240
N_EDGES = 320000
NC = 2
NS = 16
NR = 5120
ACC_ROWS = 5376
ART = ACC_ROWS // NS
CHUNK = 128
NCH = 157
EPT = NCH * CHUNK
EPAD = NS * EPT
RPT = NPAD // NS
BN = 256
NB = NPAD // BN
NBR = NR // BN


@functools.cache
def _mesh():
    return plsc.VectorSubcoreMesh(core_axis_name="c", subcore_axis_name="s",
                                  num_cores=NC, num_subcores=NS)


def _agg_pass(hs_hbm, zero_hbm, out_slc, p, src_slc, dst_slc, cnt_hbm,
              src_v, dst_v, rows0_v, rows1_v, cnt_v, acc_sh, sem0, sem1,
              barrier_before, trip_split=None):
    s = lax.axis_index("s")
    if barrier_before:
        plsc.subcore_barrier()
    pltpu.sync_copy(zero_hbm, acc_sh.at[pl.ds(s * ART, ART)])
    pltpu.sync_copy(src_slc, src_v)
    pltpu.sync_copy(dst_slc, dst_v)
    pltpu.sync_copy(cnt_hbm.at[s], cnt_v)
    cnt = jnp.max(cnt_v[pl.ds(p * 16, 16)])
    trips = (cnt + CHUNK - 1) // CHUNK
    if trip_split is None:
        lo, hi = jnp.int32(0), trips
    else:
        half = trips // 2
        lo = jnp.where(trip_split == 0, 0, half)
        hi = jnp.where(trip_split == 0, half, trips)
    plsc.subcore_barrier()

    @pl.when(hi > lo)
    def _():
        pltpu.async_copy(hs_hbm.at[src_v.at[lo]], rows0_v, sem0)

    def body(j, carry):
        @pl.when((j - lo) % 2 == 0)
        def _():
            @pl.when(j + 1 < hi)
            def _():
                pltpu.async_copy(hs_hbm.at[src_v.at[j + 1]], rows1_v, sem1)
            pltpu.make_async_copy(hs_hbm.at[src_v.at[j]],
                                  rows0_v, sem0).wait()
            pltpu.sync_copy(rows0_v, acc_sh.at[dst_v.at[j]], add=True)

        @pl.when((j - lo) % 2 == 1)
        def _():
            @pl.when(j + 1 < hi)
            def _():
                pltpu.async_copy(hs_hbm.at[src_v.at[j + 1]], rows0_v, sem0)
            pltpu.make_async_copy(hs_hbm.at[src_v.at[j]],
                                  rows1_v, sem1).wait()
            pltpu.sync_copy(rows1_v, acc_sh.at[dst_v.at[j]], add=True)

        return carry

    lax.fori_loop(lo, hi, body, 0)
    plsc.subcore_barrier()
    pltpu.sync_copy(acc_sh.at[pl.ds(s * ART, ART)], out_slc)


def _agg_scratch():
    return [
        pltpu.VMEM((NCH, CHUNK), jnp.int32),
        pltpu.VMEM((NCH, CHUNK), jnp.int32),
        pltpu.VMEM((CHUNK, 128), jnp.float32),
        pltpu.VMEM((CHUNK, 128), jnp.float32),
        pltpu.VMEM((32,), jnp.int32),
        pltpu.VMEM_SHARED((ACC_ROWS, 128), jnp.float32),
        pltpu.SemaphoreType.DMA,
        pltpu.SemaphoreType.DMA,
    ]


@functools.cache
def _make_agg1():
    @functools.partial(
        pl.kernel,
        out_type=jax.ShapeDtypeStruct((NC, 2, ACC_ROWS, 128), jnp.float32),
        mesh=_mesh(),
        compiler_params=pltpu.CompilerParams(needs_layout_passes=False),
        scratch_types=_agg_scratch(),
    )
    def agg1(hs_hbm, src_hbm, dst_hbm, cnt_hbm, zero_hbm, out_hbm,
             src_v, dst_v, rows0_v, rows1_v, cnt_v, acc_sh, sem0, sem1):
        c = lax.axis_index("c")
        s = lax.axis_index("s")
        for p in range(2):
            _agg_pass(hs_hbm, zero_hbm,
                      out_hbm.at[c, p, pl.ds(s * ART, ART)],
                      p, src_hbm.at[c, p, s], dst_hbm.at[p, s], cnt_hbm,
                      src_v, dst_v, rows0_v, rows1_v, cnt_v, acc_sh,
                      sem0, sem1, barrier_before=(p > 0))

    return agg1


@functools.cache
def _make_agg2():
    @functools.partial(
        pl.kernel,
        out_type=jax.ShapeDtypeStruct((NC, ACC_ROWS, 128), jnp.float32),
        mesh=_mesh(),
        compiler_params=pltpu.CompilerParams(needs_layout_passes=False),
        scratch_types=_agg_scratch(),
    )
    def agg2(hs_hbm, src_hbm, dst_hbm, cnt_hbm, zero_hbm, out_hbm,
             src_v, dst_v, rows0_v, rows1_v, cnt_v, acc_sh, sem0, sem1):
        c = lax.axis_index("c")
        s = lax.axis_index("s")
        _agg_pass(hs_hbm, zero_hbm,
                  out_hbm.at[c, pl.ds(s * ART, ART)],
                  c, src_hbm.at[c, s], dst_hbm.at[c, s], cnt_hbm,
                  src_v, dst_v, rows0_v, rows1_v, cnt_v, acc_sh,
                  sem0, sem1, barrier_before=False)

    return agg2


@functools.cache
def _make_part():
    @functools.partial(
        pl.kernel,
        out_type=(jax.ShapeDtypeStruct((2, NS, EPT), jnp.int32),
                  jax.ShapeDtypeStruct((2, NS, EPT), jnp.int32),
                  jax.ShapeDtypeStruct((NS, 32), jnp.int32)),
        mesh=_mesh(),
        compiler_params=pltpu.CompilerParams(needs_layout_passes=False),
        scratch_types=[
            pltpu.VMEM((EPT,), jnp.int32),
            pltpu.VMEM((EPT,), jnp.int32),
            pltpu.VMEM((EPT,), jnp.int32),
            pltpu.VMEM((EPT,), jnp.int32),
            pltpu.VMEM((EPT,), jnp.int32),
            pltpu.VMEM((EPT,), jnp.int32),
            pltpu.VMEM((32,), jnp.int32),
        ],
    )
    def part_kernel(src_hbm, dst_hbm, zfill_hbm, dfill_hbm,
                    osrc_hbm, odst_hbm, ocnt_hbm,
                    src_v, dst_v, bs0, bs1, bd0, bd1, cnt_v):
        c = lax.axis_index("c")
        s = lax.axis_index("s")

        @pl.when(c == 0)
        def _():
            pltpu.sync_copy(src_hbm.at[s], src_v)
            pltpu.sync_copy(dst_hbm.at[s], dst_v)
            pltpu.sync_copy(zfill_hbm, bs0)
            pltpu.sync_copy(zfill_hbm, bs1)
            pltpu.sync_copy(dfill_hbm, bd0)
            pltpu.sync_copy(dfill_hbm, bd1)

            def body(i, cnts):
                c0, c1 = cnts
                sv = src_v[pl.ds(i * 16, 16)]
                dv = dst_v[pl.ds(i * 16, 16)]
                m0 = dv < NR
                m1 = jnp.logical_not(m0)
                plsc.store_compressed(bs0.at[pl.ds(c0, 16)], sv, mask=m0)
                plsc.store_compressed(bd0.at[pl.ds(c0, 16)], dv, mask=m0)
                plsc.store_compressed(bs1.at[pl.ds(c1, 16)], sv, mask=m1)
                plsc.store_compressed(bd1.at[pl.ds(c1, 16)], dv - NR,
                                      mask=m1)
                n0 = jnp.sum(m0.astype(jnp.int32))
                return (c0 + n0, c1 + (16 - n0))

            c0, c1 = lax.fori_loop(0, EPT // 16, body,
                                   (jnp.int32(0), jnp.int32(0)))
            cnt_v[pl.ds(0, 16)] = jnp.full((16,), 1, jnp.int32) * c0
            cnt_v[pl.ds(16, 16)] = jnp.full((16,), 1, jnp.int32) * c1
            pltpu.sync_copy(bs0, osrc_hbm.at[0, s])
            pltpu.sync_copy(bs1, osrc_hbm.at[1, s])
            pltpu.sync_copy(bd0, odst_hbm.at[0, s])
            pltpu.sync_copy(bd1, odst_hbm.at[1, s])
            pltpu.sync_copy(cnt_v, ocnt_hbm.at[s])

    return part_kernel


@functools.cache
def _make_deg():
    @functools.partial(
        pl.kernel,
        out_type=jax.ShapeDtypeStruct((NPAD,), jnp.float32),
        mesh=_mesh(),
        compiler_params=pltpu.CompilerParams(needs_layout_passes=False),
        scratch_types=[
            pltpu.VMEM((EPT,), jnp.int32),
            pltpu.VMEM((NPAD,), jnp.float32),
            pltpu.VMEM((NS, RPT), jnp.float32),
            pltpu.VMEM((RPT,), jnp.float32),
            pltpu.VMEM_SHARED((NS, NPAD), jnp.float32),
        ],
    )
    def deg_kernel(dst_hbm, zero_hbm, out_hbm, dst_v, ldeg_v, mrg_v, mout_v,
                   sh):
        c = lax.axis_index("c")
        s = lax.axis_index("s")

        @pl.when(c == 0)
        def _():
            pltpu.sync_copy(dst_hbm.at[s], dst_v)
            pltpu.sync_copy(zero_hbm, ldeg_v)
            ones16 = jnp.full((16,), 1.0, jnp.float32)

            def body(i, carry):
                idx = dst_v[pl.ds(i * 16, 16)]
                plsc.addupdate_scatter(ldeg_v, [idx], ones16)
                return carry

            lax.fori_loop(0, EPT // 16, body, 0)
            pltpu.sync_copy(ldeg_v, sh.at[s])
            plsc.subcore_barrier()
            pltpu.sync_copy(sh.at[:, pl.ds(s * RPT, RPT)], mrg_v)
            for k in range(RPT // 16):
                acc = None
                for t in range(NS):
                    row = mrg_v[t, pl.ds(k * 16, 16)]
                    acc = row if acc is None else acc + row
                mout_v[pl.ds(k * 16, 16)] = acc
            pltpu.sync_copy(mout_v, out_hbm.at[pl.ds(s * RPT, RPT)])

    return deg_kernel


def _tc1(x_p, w1h, deg):
    def body(x_ref, w_ref, deg_ref, o_ref):
        dis = lax.rsqrt(deg_ref[...] + 1.0)
        h = jnp.dot(x_ref[...], w_ref[0], preferred_element_type=jnp.float32)
        o_ref[...] = h * dis

    return pl.pallas_call(
        body,
        grid=(NB, 2),
        in_specs=[
            pl.BlockSpec((BN, 128), lambda i, j: (i, 0)),
            pl.BlockSpec((1, 128, 128), lambda i, j: (j, 0, 0)),
            pl.BlockSpec((BN, 1), lambda i, j: (i, 0)),
        ],
        out_specs=pl.BlockSpec((BN, 128), lambda i, j: (j * NB + i, 0)),
        out_shape=jax.ShapeDtypeStruct((2 * NPAD, 128), jnp.float32),
    )(x_p, w1h, deg)


def _tc2(agg1, hs1, deg, b1, w2):
    def body(a_lo, a_hi, h_lo, h_hi, deg_ref, b_ref, w_ref, o_ref):
        dis = lax.rsqrt(deg_ref[...] + 1.0)
        b = b_ref[...]
        w = w_ref[...]
        r_lo = jnp.maximum(dis * (a_lo[0, 0] + h_lo[...]) + b[:, :128], 0.0)
        r_hi = jnp.maximum(dis * (a_hi[0, 0] + h_hi[...]) + b[:, 128:], 0.0)
        h2 = (jnp.dot(r_lo, w[:128], preferred_element_type=jnp.float32)
              + jnp.dot(r_hi, w[128:], preferred_element_type=jnp.float32))
        o_ref[...] = h2 * dis

    return pl.pallas_call(
        body,
        grid=(NB,),
        in_specs=[
            pl.BlockSpec((1, 1, BN, 128), lambda i: (0, i // NBR, i % NBR, 0)),
            pl.BlockSpec((1, 1, BN, 128), lambda i: (1, i // NBR, i % NBR, 0)),
            pl.BlockSpec((BN, 128), lambda i: (i, 0)),
            pl.BlockSpec((BN, 128), lambda i: (NB + i, 0)),
            pl.BlockSpec((BN, 1), lambda i: (i, 0)),
            pl.BlockSpec((1, 256), lambda i: (0, 0)),
            pl.BlockSpec((256, 128), lambda i: (0, 0)),
        ],
        out_specs=pl.BlockSpec((BN, 128), lambda i: (i, 0)),
        out_shape=jax.ShapeDtypeStruct((NPAD, 128), jnp.float32),
    )(agg1, agg1, hs1, hs1, deg, b1, w2)


def _tc3(agg2, hs2, deg, b2):
    def body(a_ref, h_ref, deg_ref, b_ref, o_ref):
        dis = lax.rsqrt(deg_ref[...] + 1.0)
        o_ref[...] = dis * (a_ref[0] + h_ref[...]) + b_ref[...]

    return pl.pallas_call(
        body,
        grid=(NB,),
        in_specs=[
            pl.BlockSpec((1, BN, 128), lambda i: (i // NBR, i % NBR, 0)),
            pl.BlockSpec((BN, 128), lambda i: (i, 0)),
            pl.BlockSpec((BN, 1), lambda i: (i, 0)),
            pl.BlockSpec((1, 128), lambda i: (0, 0)),
        ],
        out_specs=pl.BlockSpec((BN, 128), lambda i: (i, 0)),
        out_shape=jax.ShapeDtypeStruct((NPAD, 128), jnp.float32),
    )(agg2, hs2, deg, b2)


def kernel(x, edge_index, W1, b1, W2, b2):
    x = x.astype(jnp.float32)
    src = edge_index[0].astype(jnp.int32)
    dst = edge_index[1].astype(jnp.int32)
    pad_e = EPAD - N_EDGES
    srcp = jnp.concatenate([src, jnp.zeros((pad_e,), jnp.int32)])
    dstp = jnp.concatenate([dst, jnp.full((pad_e,), N_NODES, jnp.int32)])
    x_p = jnp.pad(x, ((0, NPAD - N_NODES), (0, 0)))
    zero_acc = jnp.zeros((ART, 128), jnp.float32)

    src_l, dst_l, counts = _make_part()(
        srcp.reshape(NS, EPT), dstp.reshape(NS, EPT),
        jnp.zeros((EPT,), jnp.int32), jnp.full((EPT,), NR, jnp.int32))
    src1 = (src_l[None] + jnp.array([0, NPAD], jnp.int32)[:, None, None, None]
            ).reshape(NC, 2, NS, NCH, CHUNK)
    src2 = src_l.reshape(2, NS, NCH, CHUNK)
    dst4 = dst_l.reshape(2, NS, NCH, CHUNK)

    deg = _make_deg()(dstp.reshape(NS, EPT),
                      jnp.zeros((NPAD,), jnp.float32))[:, None]
    hs1 = _tc1(x_p, jnp.stack([W1[:, :128], W1[:, 128:]]), deg)
    agg1 = _make_agg1()(hs1, src1, dst4, counts, zero_acc)
    hs2 = _tc2(agg1, hs1, deg, b1.reshape(1, 256), W2)
    agg2 = _make_agg2()(hs2, src2, dst4, counts, zero_acc)
    out = _tc3(agg2, hs2, deg, b2.reshape(1, 128))
    return out[:N_NODES]


# speedup vs baseline: 1.0448x; 1.0255x over previous
import functools

import jax
import jax.numpy as jnp
from jax import lax
from jax.experimental import pallas as pl
from jax.experimental.pallas import tpu as pltpu
from jax.experimental.pallas import tpu_sc as plsc

N_NODES = 10000
NPAD = 10240
N_EDGES = 320000
NC = 2
NS = 16
NR = 5120
ACC_ROWS = 5376
ART = ACC_ROWS // NS
CHUNK = 128
NCH = 157
EPT = NCH * CHUNK
EPAD = NS * EPT
RPT = NPAD // NS
BN = 256
NB = NPAD // BN
NBR = NR // BN


@functools.cache
def _mesh():
    return plsc.VectorSubcoreMesh(core_axis_name="c", subcore_axis_name="s",
                                  num_cores=NC, num_subcores=NS)


def _agg_pass(hs_hbm, zero_hbm, out_slc, p, src_slc, dst_slc, cnt_hbm,
              src_v, dst_v, rows0_v, rows1_v, cnt_v, acc_sh, sem0, sem1,
              barrier_before, trip_split=None):
    s = lax.axis_index("s")
    if barrier_before:
        plsc.subcore_barrier()
    pltpu.sync_copy(zero_hbm, acc_sh.at[pl.ds(s * ART, ART)])
    pltpu.sync_copy(src_slc, src_v)
    pltpu.sync_copy(dst_slc, dst_v)
    pltpu.sync_copy(cnt_hbm.at[s], cnt_v)
    cnt = jnp.max(cnt_v[pl.ds(p * 16, 16)])
    trips = (cnt + CHUNK - 1) // CHUNK
    if trip_split is None:
        lo, hi = jnp.int32(0), trips
    else:
        half = trips // 2
        lo = jnp.where(trip_split == 0, 0, half)
        hi = jnp.where(trip_split == 0, half, trips)
    plsc.subcore_barrier()

    @pl.when(hi > lo)
    def _():
        pltpu.async_copy(hs_hbm.at[src_v.at[lo]], rows0_v, sem0)

    def body(j, carry):
        @pl.when((j - lo) % 2 == 0)
        def _():
            @pl.when(j + 1 < hi)
            def _():
                pltpu.async_copy(hs_hbm.at[src_v.at[j + 1]], rows1_v, sem1)
            pltpu.make_async_copy(hs_hbm.at[src_v.at[j]],
                                  rows0_v, sem0).wait()
            pltpu.sync_copy(rows0_v, acc_sh.at[dst_v.at[j]], add=True)

        @pl.when((j - lo) % 2 == 1)
        def _():
            @pl.when(j + 1 < hi)
            def _():
                pltpu.async_copy(hs_hbm.at[src_v.at[j + 1]], rows0_v, sem0)
            pltpu.make_async_copy(hs_hbm.at[src_v.at[j]],
                                  rows1_v, sem1).wait()
            pltpu.sync_copy(rows1_v, acc_sh.at[dst_v.at[j]], add=True)

        return carry

    lax.fori_loop(lo, hi, body, 0)
    plsc.subcore_barrier()
    pltpu.sync_copy(acc_sh.at[pl.ds(s * ART, ART)], out_slc)


def _agg_scratch():
    return [
        pltpu.VMEM((NCH, CHUNK), jnp.int32),
        pltpu.VMEM((NCH, CHUNK), jnp.int32),
        pltpu.VMEM((CHUNK, 128), jnp.float32),
        pltpu.VMEM((CHUNK, 128), jnp.float32),
        pltpu.VMEM((32,), jnp.int32),
        pltpu.VMEM_SHARED((ACC_ROWS, 128), jnp.float32),
        pltpu.SemaphoreType.DMA,
        pltpu.SemaphoreType.DMA,
    ]


@functools.cache
def _make_agg1():
    @functools.partial(
        pl.kernel,
        out_type=jax.ShapeDtypeStruct((NC, 2, ACC_ROWS, 128), jnp.float32),
        mesh=_mesh(),
        compiler_params=pltpu.CompilerParams(needs_layout_passes=False),
        scratch_types=_agg_scratch(),
    )
    def agg1(hs_hbm, src_hbm, dst_hbm, cnt_hbm, zero_hbm, out_hbm,
             src_v, dst_v, rows0_v, rows1_v, cnt_v, acc_sh, sem0, sem1):
        c = lax.axis_index("c")
        s = lax.axis_index("s")
        for p in range(2):
            _agg_pass(hs_hbm, zero_hbm,
                      out_hbm.at[c, p, pl.ds(s * ART, ART)],
                      p, src_hbm.at[c, p, s], dst_hbm.at[p, s], cnt_hbm,
                      src_v, dst_v, rows0_v, rows1_v, cnt_v, acc_sh,
                      sem0, sem1, barrier_before=(p > 0))

    return agg1


@functools.cache
def _make_agg2():
    @functools.partial(
        pl.kernel,
        out_type=jax.ShapeDtypeStruct((NC, ACC_ROWS, 128), jnp.float32),
        mesh=_mesh(),
        compiler_params=pltpu.CompilerParams(needs_layout_passes=False),
        scratch_types=_agg_scratch(),
    )
    def agg2(hs_hbm, src_hbm, dst_hbm, cnt_hbm, zero_hbm, out_hbm,
             src_v, dst_v, rows0_v, rows1_v, cnt_v, acc_sh, sem0, sem1):
        c = lax.axis_index("c")
        s = lax.axis_index("s")
        _agg_pass(hs_hbm, zero_hbm,
                  out_hbm.at[c, pl.ds(s * ART, ART)],
                  c, src_hbm.at[c, s], dst_hbm.at[c, s], cnt_hbm,
                  src_v, dst_v, rows0_v, rows1_v, cnt_v, acc_sh,
                  sem0, sem1, barrier_before=False)

    return agg2


@functools.cache
def _make_prep():
    @functools.partial(
        pl.kernel,
        out_type=(jax.ShapeDtypeStruct((2, NS, EPT), jnp.int32),
                  jax.ShapeDtypeStruct((2, NS, EPT), jnp.int32),
                  jax.ShapeDtypeStruct((NS, 32), jnp.int32),
                  jax.ShapeDtypeStruct((NPAD,), jnp.float32)),
        mesh=_mesh(),
        compiler_params=pltpu.CompilerParams(needs_layout_passes=False),
        scratch_types=[
            pltpu.VMEM((EPT + 16,), jnp.int32),
            pltpu.VMEM((EPT + 16,), jnp.int32),
            pltpu.VMEM((EPT,), jnp.int32),
            pltpu.VMEM((EPT,), jnp.int32),
            pltpu.VMEM((32,), jnp.int32),
            pltpu.VMEM((NPAD,), jnp.float32),
            pltpu.VMEM((NS, RPT), jnp.float32),
            pltpu.VMEM((RPT,), jnp.float32),
            pltpu.VMEM_SHARED((NS, NPAD), jnp.float32),
        ],
    )
    def prep_kernel(src_hbm, dst_hbm, zfill_hbm, dfill_hbm, zdeg_hbm,
                    osrc_hbm, odst_hbm, ocnt_hbm, odeg_hbm,
                    src_v, dst_v, bs0, bd0, cnt_v, ldeg_v, mrg_v, mout_v, sh):
        c = lax.axis_index("c")
        s = lax.axis_index("s")

        @pl.when(c == 0)
        def _():
            pltpu.sync_copy(src_hbm.at[s], src_v.at[pl.ds(0, EPT)])
            pltpu.sync_copy(dst_hbm.at[s], dst_v.at[pl.ds(0, EPT)])
            pltpu.sync_copy(zfill_hbm, bs0)
            pltpu.sync_copy(dfill_hbm, bd0)

            def body(i, cnts):
                c0, c1 = cnts
                sv = src_v[pl.ds(i * 16, 16)]
                dv = dst_v[pl.ds(i * 16, 16)]
                m0 = dv < NR
                m1 = jnp.logical_not(m0)
                plsc.store_compressed(bs0.at[pl.ds(c0, 16)], sv, mask=m0)
                plsc.store_compressed(bd0.at[pl.ds(c0, 16)], dv, mask=m0)
                plsc.store_compressed(src_v.at[pl.ds(c1, 16)], sv, mask=m1)
                plsc.store_compressed(dst_v.at[pl.ds(c1, 16)], dv - NR,
                                      mask=m1)
                n0 = jnp.sum(m0.astype(jnp.int32))
                return (c0 + n0, c1 + (16 - n0))

            c0, c1 = lax.fori_loop(0, EPT // 16, body,
                                   (jnp.int32(0), jnp.int32(0)))
            pad_end = ((c1 + CHUNK - 1) // CHUNK) * CHUNK
            zeros16 = jnp.full((16,), 0, jnp.int32)
            dumps16 = jnp.full((16,), NR, jnp.int32)

            def pad_body(k, carry):
                @pl.when(c1 + k * 16 < pad_end)
                def _():
                    src_v[pl.ds(c1 + k * 16, 16)] = zeros16
                    dst_v[pl.ds(c1 + k * 16, 16)] = dumps16
                return carry

            lax.fori_loop(0, 8, pad_body, 0)
            cnt_v[pl.ds(0, 16)] = jnp.full((16,), 1, jnp.int32) * c0
            cnt_v[pl.ds(16, 16)] = jnp.full((16,), 1, jnp.int32) * c1
            pltpu.sync_copy(bs0, osrc_hbm.at[0, s])
            pltpu.sync_copy(src_v.at[pl.ds(0, EPT)], osrc_hbm.at[1, s])
            pltpu.sync_copy(bd0, odst_hbm.at[0, s])
            pltpu.sync_copy(dst_v.at[pl.ds(0, EPT)], odst_hbm.at[1, s])
            pltpu.sync_copy(cnt_v, ocnt_hbm.at[s])

        @pl.when(c == 1)
        def _():
            pltpu.sync_copy(dst_hbm.at[s], dst_v.at[pl.ds(0, EPT)])
            pltpu.sync_copy(zdeg_hbm, ldeg_v)
            ones16 = jnp.full((16,), 1.0, jnp.float32)

            def body(i, carry):
                idx = dst_v[pl.ds(i * 16, 16)]
                plsc.addupdate_scatter(ldeg_v, [idx], ones16)
                return carry

            lax.fori_loop(0, EPT // 16, body, 0)
            pltpu.sync_copy(ldeg_v, sh.at[s])
            plsc.subcore_barrier()
            pltpu.sync_copy(sh.at[:, pl.ds(s * RPT, RPT)], mrg_v)
            for k in range(RPT // 16):
                acc = None
                for t in range(NS):
                    row = mrg_v[t, pl.ds(k * 16, 16)]
                    acc = row if acc is None else acc + row
                mout_v[pl.ds(k * 16, 16)] = acc
            pltpu.sync_copy(mout_v, odeg_hbm.at[pl.ds(s * RPT, RPT)])

    return prep_kernel


def _tc1(x_p, w1h, deg):
    def body(x_ref, w_ref, deg_ref, o_ref):
        dis = lax.rsqrt(deg_ref[...] + 1.0)
        h = jnp.dot(x_ref[...], w_ref[0], preferred_element_type=jnp.float32)
        o_ref[...] = h * dis

    return pl.pallas_call(
        body,
        grid=(NB, 2),
        in_specs=[
            pl.BlockSpec((BN, 128), lambda i, j: (i, 0)),
            pl.BlockSpec((1, 128, 128), lambda i, j: (j, 0, 0)),
            pl.BlockSpec((BN, 1), lambda i, j: (i, 0)),
        ],
        out_specs=pl.BlockSpec((BN, 128), lambda i, j: (j * NB + i, 0)),
        out_shape=jax.ShapeDtypeStruct((2 * NPAD, 128), jnp.float32),
    )(x_p, w1h, deg)


def _tc2(agg1, hs1, deg, b1, w2):
    def body(a_lo, a_hi, h_lo, h_hi, deg_ref, b_ref, w_ref, o_ref):
        dis = lax.rsqrt(deg_ref[...] + 1.0)
        b = b_ref[...]
        w = w_ref[...]
        r_lo = jnp.maximum(dis * (a_lo[0, 0] + h_lo[...]) + b[:, :128], 0.0)
        r_hi = jnp.maximum(dis * (a_hi[0, 0] + h_hi[...]) + b[:, 128:], 0.0)
        h2 = (jnp.dot(r_lo, w[:128], preferred_element_type=jnp.float32)
              + jnp.dot(r_hi, w[128:], preferred_element_type=jnp.float32))
        o_ref[...] = h2 * dis

    return pl.pallas_call(
        body,
        grid=(NB,),
        in_specs=[
            pl.BlockSpec((1, 1, BN, 128), lambda i: (0, i // NBR, i % NBR, 0)),
            pl.BlockSpec((1, 1, BN, 128), lambda i: (1, i // NBR, i % NBR, 0)),
            pl.BlockSpec((BN, 128), lambda i: (i, 0)),
            pl.BlockSpec((BN, 128), lambda i: (NB + i, 0)),
            pl.BlockSpec((BN, 1), lambda i: (i, 0)),
            pl.BlockSpec((1, 256), lambda i: (0, 0)),
            pl.BlockSpec((256, 128), lambda i: (0, 0)),
        ],
        out_specs=pl.BlockSpec((BN, 128), lambda i: (i, 0)),
        out_shape=jax.ShapeDtypeStruct((NPAD, 128), jnp.float32),
    )(agg1, agg1, hs1, hs1, deg, b1, w2)


def _tc3(agg2, hs2, deg, b2):
    def body(a_ref, h_ref, deg_ref, b_ref, o_ref):
        dis = lax.rsqrt(deg_ref[...] + 1.0)
        o_ref[...] = dis * (a_ref[0] + h_ref[...]) + b_ref[...]

    return pl.pallas_call(
        body,
        grid=(NB,),
        in_specs=[
            pl.BlockSpec((1, BN, 128), lambda i: (i // NBR, i % NBR, 0)),
            pl.BlockSpec((BN, 128), lambda i: (i, 0)),
            pl.BlockSpec((BN, 1), lambda i: (i, 0)),
            pl.BlockSpec((1, 128), lambda i: (0, 0)),
        ],
        out_specs=pl.BlockSpec((BN, 128), lambda i: (i, 0)),
        out_shape=jax.ShapeDtypeStruct((NPAD, 128), jnp.float32),
    )(agg2, hs2, deg, b2)


def kernel(x, edge_index, W1, b1, W2, b2):
    x = x.astype(jnp.float32)
    src = edge_index[0].astype(jnp.int32)
    dst = edge_index[1].astype(jnp.int32)
    pad_e = EPAD - N_EDGES
    srcp = jnp.concatenate([src, jnp.zeros((pad_e,), jnp.int32)])
    dstp = jnp.concatenate([dst, jnp.full((pad_e,), N_NODES, jnp.int32)])
    x_p = jnp.pad(x, ((0, NPAD - N_NODES), (0, 0)))
    zero_acc = jnp.zeros((ART, 128), jnp.float32)

    src_l, dst_l, counts, degv = _make_prep()(
        srcp.reshape(NS, EPT), dstp.reshape(NS, EPT),
        jnp.zeros((EPT,), jnp.int32), jnp.full((EPT,), NR, jnp.int32),
        jnp.zeros((NPAD,), jnp.float32))
    src1 = (src_l[None] + jnp.array([0, NPAD], jnp.int32)[:, None, None, None]
            ).reshape(NC, 2, NS, NCH, CHUNK)
    src2 = src_l.reshape(2, NS, NCH, CHUNK)
    dst4 = dst_l.reshape(2, NS, NCH, CHUNK)

    deg = degv[:, None]
    hs1 = _tc1(x_p, jnp.stack([W1[:, :128], W1[:, 128:]]), deg)
    agg1 = _make_agg1()(hs1, src1, dst4, counts, zero_acc)
    hs2 = _tc2(agg1, hs1, deg, b1.reshape(1, 256), W2)
    agg2 = _make_agg2()(hs2, src2, dst4, counts, zero_acc)
    out = _tc3(agg2, hs2, deg, b2.reshape(1, 128))
    return out[:N_NODES]
